# baseline (speedup 1.0000x reference)
import functools

import jax
import jax.numpy as jnp
from jax import lax
from jax.experimental import pallas as pl
from jax.experimental.pallas import tpu as pltpu
from jax.experimental.pallas import tpu_sc as plsc

NC = 2
NS = 16
NW = NC * NS
BLK = 128
NB = 4


@functools.partial(jax.jit, static_argnames=("n", "fp", "bpt", "packed_tbl"))
def _sc_segment(table, pc3, w2, *, n, fp, bpt, packed_tbl):
    n_per_tile = -(-(n // NS) // 8) * 8
    n_last = n - n_per_tile * (NS - 1)
    sh = (n - 1).bit_length()
    ft = fp // 2 if packed_tbl else fp
    ftp = -(-ft // 8) * 8 if packed_tbl else ft
    mesh = plsc.VectorSubcoreMesh(
        core_axis_name="c", subcore_axis_name="s", num_cores=NC, num_subcores=NS
    )
    tdt = jnp.int32 if packed_tbl else jnp.float32

    @functools.partial(
        pl.kernel,
        mesh=mesh,
        compiler_params=pltpu.CompilerParams(use_tc_tiling_on_sc=False),
        out_type=jax.ShapeDtypeStruct((NC, n, fp), jnp.float32),
        scratch_types=[
            pltpu.VMEM((bpt, BLK), jnp.int32),
            pltpu.VMEM((bpt, BLK), jnp.int32),
            pltpu.VMEM((bpt, BLK), jnp.int32),
            pltpu.VMEM((bpt, BLK), jnp.float32),
            pltpu.VMEM((NB, BLK, ftp), tdt),
            pltpu.VMEM((NB, BLK, fp), jnp.float32),
            pltpu.VMEM_SHARED((n, fp), jnp.float32),
            pltpu.VMEM_SHARED((n, ftp), tdt),
        ] + [pltpu.SemaphoreType.DMA] * (2 * NB),
    )
    def seg(table_h, pc_h, w_h, out_h, pk_v, col_v, row_v, w_v, rows_v,
            msg_v, acc, tbl_s, *bufsems):
        gsems = bufsems[:NB]
        ssems = bufsems[NB:]
        c = lax.axis_index("c")
        s = lax.axis_index("s")
        wid = c * NS + s

        pltpu.sync_copy(pc_h.at[wid], pk_v)
        pltpu.sync_copy(w_h.at[wid], w_v)

        cmask = jnp.full((16,), (1 << sh) - 1, jnp.int32)
        cshift = jnp.full((16,), sh, jnp.int32)

        def ubody(i, carry):
            for j in range(BLK // 16):
                sl = pl.ds(j * 16, 16)
                v = pk_v[i, sl]
                col_v[i, sl] = v & cmask
                row_v[i, sl] = lax.shift_right_logical(v, cshift)
            return carry

        lax.fori_loop(0, bpt, ubody, 0)

        zero16 = jnp.zeros((16,), jnp.float32)
        zbuf = msg_v.at[0]

        def zbody(i, carry):
            for j in range(fp // 16):
                zbuf[i, pl.ds(j * 16, 16)] = zero16
            if fp % 16:
                zbuf[i, pl.ds(fp - 16, 16)] = zero16
            return carry

        lax.fori_loop(0, BLK, zbody, 0)
        base = pl.multiple_of(s * n_per_tile, 8)

        def zero_and_stage(rows):
            for k in range(0, rows, BLK):
                sz = min(BLK, rows - k)
                pltpu.sync_copy(zbuf.at[pl.ds(0, sz)],
                                acc.at[pl.ds(base + k, sz)])
            pltpu.sync_copy(table_h.at[pl.ds(base, rows)],
                            tbl_s.at[pl.ds(base, rows)])

        @pl.when(s < NS - 1)
        def _():
            zero_and_stage(n_per_tile)

        @pl.when(s == NS - 1)
        def _():
            zero_and_stage(n_last)

        plsc.subcore_barrier()

        def gather_start(blk, b):
            pltpu.async_copy(tbl_s.at[col_v.at[blk]], rows_v.at[b], gsems[b])

        def gather_wait(blk, b):
            pltpu.make_async_copy(
                tbl_s.at[col_v.at[blk]], rows_v.at[b], gsems[b]
            ).wait()

        def scatter_start(blk, b):
            pltpu.async_copy(
                msg_v.at[b], acc.at[row_v.at[blk]], ssems[b], add=True
            )

        def scatter_wait(blk, b):
            pltpu.make_async_copy(
                msg_v.at[b], acc.at[row_v.at[blk]], ssems[b]
            ).wait()

        c16 = jnp.full((16,), 16, jnp.int32)
        chi = jnp.full((16,), -65536, jnp.int32)

        def scale_packed(b, e, wsplat):
            v1 = rows_v[b, e, pl.ds(0, 16)]
            v2 = rows_v[b, e, pl.ds(ft - 16, 16)]
            lo1 = lax.bitcast_convert_type(lax.shift_left(v1, c16),
                                           jnp.float32)
            hi1 = lax.bitcast_convert_type(v1 & chi, jnp.float32)
            lo2 = lax.bitcast_convert_type(lax.shift_left(v2, c16),
                                           jnp.float32)
            hi2 = lax.bitcast_convert_type(v2 & chi, jnp.float32)
            msg_v[b, e, pl.ds(0, 16)] = lo1 * wsplat
            msg_v[b, e, pl.ds(ft, 16)] = hi1 * wsplat
            msg_v[b, e, pl.ds(ft - 16, 16)] = lo2 * wsplat
            msg_v[b, e, pl.ds(fp - 16, 16)] = hi2 * wsplat

        def scale_plain(b, e, wsplat):
            for j in range(fp // 16):
                sl = pl.ds(j * 16, 16)
                msg_v[b, e, sl] = rows_v[b, e, sl] * wsplat
            if fp % 16:
                sl = pl.ds(fp - 16, 16)
                msg_v[b, e, sl] = rows_v[b, e, sl] * wsplat

        def compute(blk, b):
            def gbody(g, c2):
                wv = w_v[blk, pl.ds(g * 16, 16)]
                for l in range(16):
                    e = g * 16 + l
                    wsplat = jnp.full((16,), wv[l], jnp.float32)
                    if packed_tbl:
                        scale_packed(b, e, wsplat)
                    else:
                        scale_plain(b, e, wsplat)
                return c2

            lax.fori_loop(0, BLK // 16, gbody, 0)

        gather_start(0, 0)
        gather_start(1, 1)
        for blk in range(NB):
            if blk >= 2:
                scatter_wait(blk - 2, (blk - 2) % NB)
            gather_start(blk + 2, (blk + 2) % NB)
            gather_wait(blk, blk)
            compute(blk, blk)
            scatter_start(blk, blk)

        def mbody(m, carry):
            for b in range(NB):
                blk = m * NB + b
                scatter_wait(blk - 2, (b - 2) % NB)
                gather_start(blk + 2, (b + 2) % NB)
                gather_wait(blk, b)
                compute(blk, b)
                scatter_start(blk, b)
            return carry

        lax.fori_loop(1, bpt // NB - 1, mbody, 0)

        for blk in range(bpt - NB, bpt):
            b = blk % NB
            scatter_wait(blk - 2, (blk - 2) % NB)
            if blk + 2 < bpt:
                gather_start(blk + 2, (blk + 2) % NB)
            gather_wait(blk, b)
            compute(blk, b)
            scatter_start(blk, b)
        scatter_wait(bpt - 2, (bpt - 2) % NB)
        scatter_wait(bpt - 1, (bpt - 1) % NB)
        plsc.subcore_barrier()

        @pl.when(s < NS - 1)
        def _():
            pltpu.sync_copy(acc.at[pl.ds(base, n_per_tile)],
                            out_h.at[c, pl.ds(base, n_per_tile)])

        @pl.when(s == NS - 1)
        def _():
            pltpu.sync_copy(acc.at[pl.ds(base, n_last)],
                            out_h.at[c, pl.ds(base, n_last)])

    return seg(table, pc3, w2)


def _mm_bias(x, w, b):
    n, d = x.shape
    f = w.shape[1]
    br = 2000
    grid = n // br

    def body(x_ref, w_ref, b_ref, o_ref):
        o_ref[...] = (
            jnp.dot(x_ref[...], w_ref[...], preferred_element_type=jnp.float32)
            + b_ref[...]
        )

    return pl.pallas_call(
        body,
        grid=(grid,),
        in_specs=[
            pl.BlockSpec((br, d), lambda i: (i, 0)),
            pl.BlockSpec((d, f), lambda i: (0, 0)),
            pl.BlockSpec((1, f), lambda i: (0, 0)),
        ],
        out_specs=pl.BlockSpec((br, f), lambda i: (i, 0)),
        out_shape=jax.ShapeDtypeStruct((n, f), jnp.float32),
    )(x, w, b)


def _relu_sum_mm_packed(p, w, b):
    _, n, f1 = p.shape
    f2 = w.shape[1]
    ftp = -(-(f2 // 2) // 8) * 8
    br = 2000
    grid = n // br

    def body(p_ref, w_ref, b_ref, o_ref):
        h = jnp.maximum(p_ref[0] + p_ref[1], 0.0)
        hw = jnp.dot(h, w_ref[...], preferred_element_type=jnp.float32) \
            + b_ref[...]
        lo = lax.bitcast_convert_type(
            hw[:, : f2 // 2].astype(jnp.bfloat16), jnp.uint16
        ).astype(jnp.uint32)
        hi = lax.bitcast_convert_type(
            hw[:, f2 // 2:].astype(jnp.bfloat16), jnp.uint16
        ).astype(jnp.uint32)
        pk = lax.bitcast_convert_type(lo | (hi << 16), jnp.int32)
        fpd = o_ref.shape[1] - pk.shape[1]
        if fpd:
            pk = jnp.concatenate(
                [pk, jnp.zeros((pk.shape[0], fpd), jnp.int32)], axis=1)
        o_ref[...] = pk

    return pl.pallas_call(
        body,
        grid=(grid,),
        in_specs=[
            pl.BlockSpec((2, br, f1), lambda i: (0, i, 0)),
            pl.BlockSpec((f1, f2), lambda i: (0, 0)),
            pl.BlockSpec((1, f2), lambda i: (0, 0)),
        ],
        out_specs=pl.BlockSpec((br, ftp), lambda i: (i, 0)),
        out_shape=jax.ShapeDtypeStruct((n, ftp), jnp.int32),
    )(p, w, b)


def _relu_sum_slice(p, f_out):
    _, n, fp = p.shape
    br = 2000
    grid = n // br

    def body(p_ref, o_ref):
        h = jnp.maximum(p_ref[0] + p_ref[1], 0.0)
        o_ref[...] = h[:, :f_out]

    return pl.pallas_call(
        body,
        grid=(grid,),
        in_specs=[pl.BlockSpec((2, br, fp), lambda i: (0, i, 0))],
        out_specs=pl.BlockSpec((br, f_out), lambda i: (i, 0)),
        out_shape=jax.ShapeDtypeStruct((n, f_out), jnp.float32),
    )(p)


def kernel(x, edge_index, edge_w, W1, b1, W2, b2):
    n, _ = x.shape
    h1 = W1.shape[1]
    c_out = W2.shape[1]
    fp2 = ((c_out + 7) // 8) * 8
    e = edge_index.shape[1]

    bpt = (e + NW * BLK - 1) // (NW * BLK)
    bpt = -(-bpt // NB) * NB
    epad = bpt * NW * BLK
    pad = epad - e
    sh = (n - 1).bit_length()
    packed = (edge_index[0] << sh) | edge_index[1]
    pc3 = jnp.pad(packed, (0, pad)).reshape(NW, bpt, BLK)
    w2 = jnp.pad(edge_w, (0, pad)).reshape(NW, bpt, BLK)

    W2p = jnp.pad(W2, ((0, 0), (0, fp2 - c_out)))
    b2p = jnp.pad(b2, (0, fp2 - c_out)).reshape(1, fp2)

    hw1 = _mm_bias(x, W1, b1.reshape(1, h1))
    p1 = _sc_segment(hw1, pc3, w2, n=n, fp=h1, bpt=bpt,
                     packed_tbl=False)
    hw2 = _relu_sum_mm_packed(p1, W2p, b2p)
    p2 = _sc_segment(hw2, pc3, w2, n=n, fp=fp2, bpt=bpt,
                     packed_tbl=True)
    return _relu_sum_slice(p2, c_out)
